# baseline (speedup 1.0000x reference)
import functools

import jax
import jax.numpy as jnp
from jax import lax
from jax.experimental import pallas as pl
from jax.experimental.pallas import tpu as pltpu
from jax.experimental.pallas import tpu_sc as plsc


def _make_pooling(B, CTX, D):
    info = plsc.get_sparse_core_info()
    nc, ns = info.num_cores, info.num_subcores
    nw = nc * ns
    rows_per_w = B // nw
    idx_per_w = rows_per_w * CTX
    n_chunks = -(-idx_per_w // 128)
    while idx_per_w % n_chunks or (idx_per_w // n_chunks) % 8:
        n_chunks += 1
    chunk = idx_per_w // n_chunks

    mesh = plsc.VectorSubcoreMesh(core_axis_name="c", subcore_axis_name="s")

    @functools.partial(
        pl.kernel,
        mesh=mesh,
        out_type=jax.ShapeDtypeStruct((B, D), jnp.float32),
        compiler_params=pltpu.CompilerParams(use_tc_tiling_on_sc=False),
        scratch_types=[
            pltpu.VMEM((n_chunks, chunk), jnp.int32),
            pltpu.VMEM((idx_per_w, D), jnp.float32),
            pltpu.VMEM((rows_per_w, D), jnp.float32),
            pltpu.SemaphoreType.DMA,
        ],
    )
    def pool(idx_hbm, table_hbm, out_hbm, idx_v, rows_v, acc_v, sem):
        wid = lax.axis_index("s") * nc + lax.axis_index("c")
        pltpu.sync_copy(idx_hbm.at[pl.ds(wid * n_chunks, n_chunks)], idx_v)
        copies = [
            pltpu.async_copy(
                table_hbm.at[idx_v.at[j]],
                rows_v.at[pl.ds(j * chunk, chunk)],
                sem,
            )
            for j in range(n_chunks)
        ]
        for cp in copies:
            cp.wait()

        def body(r, carry):
            base = r * CTX
            for c in range(D // 16):
                sl = pl.ds(c * 16, 16)
                acc = rows_v[base, sl]
                for k in range(1, CTX):
                    acc = acc + rows_v[base + k, sl]
                acc_v[r, sl] = acc
            return carry

        lax.fori_loop(0, rows_per_w, body, 0)
        pltpu.sync_copy(acc_v, out_hbm.at[pl.ds(wid * rows_per_w, rows_per_w)])

    def run(inputs, emb_table):
        idx = inputs.reshape(nw * n_chunks, chunk)
        return pool(idx, emb_table)

    return run


def _project(pooled, W, b, vb=2048):
    B, D = pooled.shape
    V = W.shape[0]

    def mm(p_ref, w_ref, b_ref, o_ref):
        o_ref[...] = (
            lax.dot_general(
                p_ref[...],
                w_ref[...],
                dimension_numbers=(((1,), (1,)), ((), ())),
                preferred_element_type=jnp.float32,
            )
            + b_ref[...]
        )

    return pl.pallas_call(
        mm,
        grid=(pl.cdiv(V, vb),),
        in_specs=[
            pl.BlockSpec((B, D), lambda v: (0, 0)),
            pl.BlockSpec((vb, D), lambda v: (v, 0)),
            pl.BlockSpec((1, vb), lambda v: (0, v)),
        ],
        out_specs=pl.BlockSpec((B, vb), lambda v: (0, v)),
        out_shape=jax.ShapeDtypeStruct((B, V), jnp.float32),
    )(pooled, W, b.reshape(1, V))


def kernel(inputs, emb_table, W, b):
    B, CTX = inputs.shape
    pooled = _make_pooling(B, CTX, emb_table.shape[1])(inputs, emb_table)
    return _project(pooled, W, b)


# speedup vs baseline: 2.1934x; 2.1934x over previous
import functools

import jax
import jax.numpy as jnp
from jax import lax
from jax.experimental import pallas as pl
from jax.experimental.pallas import tpu as pltpu
from jax.experimental.pallas import tpu_sc as plsc


def _make_pooling(B, CTX, D):
    info = plsc.get_sparse_core_info()
    nc, ns = info.num_cores, info.num_subcores
    nw = nc * ns
    rows_per_w = B // nw
    idx_per_w = rows_per_w * CTX
    n_chunks = -(-idx_per_w // 128)
    while idx_per_w % n_chunks or (idx_per_w // n_chunks) % 8:
        n_chunks += 1
    chunk = idx_per_w // n_chunks

    mesh = plsc.VectorSubcoreMesh(core_axis_name="c", subcore_axis_name="s")

    @functools.partial(
        pl.kernel,
        mesh=mesh,
        out_type=jax.ShapeDtypeStruct((B, D), jnp.float32),
        compiler_params=pltpu.CompilerParams(use_tc_tiling_on_sc=False),
        scratch_types=[
            pltpu.VMEM((n_chunks, chunk), jnp.int32),
            pltpu.VMEM((idx_per_w, D), jnp.float32),
            pltpu.VMEM((rows_per_w, D), jnp.float32),
            pltpu.SemaphoreType.DMA,
        ],
    )
    def pool(idx_hbm, table_hbm, out_hbm, idx_v, rows_v, acc_v, sem):
        wid = lax.axis_index("s") * nc + lax.axis_index("c")
        pltpu.sync_copy(idx_hbm.at[pl.ds(wid * n_chunks, n_chunks)], idx_v)
        copies = [
            pltpu.async_copy(
                table_hbm.at[idx_v.at[j]],
                rows_v.at[pl.ds(j * chunk, chunk)],
                sem,
            )
            for j in range(n_chunks)
        ]
        for cp in copies:
            cp.wait()

        def body(r, carry):
            base = r * CTX
            for c in range(D // 16):
                sl = pl.ds(c * 16, 16)
                acc = rows_v[base, sl]
                for k in range(1, CTX):
                    acc = acc + rows_v[base + k, sl]
                acc_v[r, sl] = acc
            return carry

        lax.fori_loop(0, rows_per_w, body, 0)
        pltpu.sync_copy(acc_v, out_hbm.at[pl.ds(wid * rows_per_w, rows_per_w)])

    def run(inputs, emb_table):
        idx = inputs.reshape(nw * n_chunks, chunk)
        return pool(idx, emb_table)

    return run


def _project(pooled, W, b, vb=2048):
    B, D = pooled.shape
    V = W.shape[0]
    w_t = W.T
    pooled_t = pooled.T
    b2 = b.reshape(V, 1)

    def mm(w_ref, p_ref, b_ref, o_ref):
        o_ref[...] = (
            lax.dot_general(
                w_ref[...],
                p_ref[...],
                dimension_numbers=(((0,), (0,)), ((), ())),
                preferred_element_type=jnp.float32,
            )
            + b_ref[...]
        )

    out_t = pl.pallas_call(
        mm,
        grid=(pl.cdiv(V, vb),),
        in_specs=[
            pl.BlockSpec((D, vb), lambda v: (0, v)),
            pl.BlockSpec((D, B), lambda v: (0, 0)),
            pl.BlockSpec((vb, 1), lambda v: (v, 0)),
        ],
        out_specs=pl.BlockSpec((vb, B), lambda v: (v, 0)),
        out_shape=jax.ShapeDtypeStruct((V, B), jnp.float32),
    )(w_t, pooled_t, b2)
    return out_t.T


def kernel(inputs, emb_table, W, b):
    B, CTX = inputs.shape
    pooled = _make_pooling(B, CTX, emb_table.shape[1])(inputs, emb_table)
    return _project(pooled, W, b)


# speedup vs baseline: 2.4283x; 1.1071x over previous
import functools

import jax
import jax.numpy as jnp
from jax import lax
from jax.experimental import pallas as pl
from jax.experimental.pallas import tpu as pltpu
from jax.experimental.pallas import tpu_sc as plsc


def _make_pooling_t(B, CTX, D, V):
    info = plsc.get_sparse_core_info()
    nc, ns = info.num_cores, info.num_subcores
    nw = nc * ns
    dims_per_w = D // nw
    n_idx = B * CTX

    mesh = plsc.VectorSubcoreMesh(core_axis_name="c", subcore_axis_name="s")

    @functools.partial(
        pl.kernel,
        mesh=mesh,
        out_type=jax.ShapeDtypeStruct((D, B), jnp.float32),
        compiler_params=pltpu.CompilerParams(
            use_tc_tiling_on_sc=False, needs_layout_passes=False
        ),
        scratch_types=[
            pltpu.VMEM((n_idx,), jnp.int32),
            pltpu.VMEM((V,), jnp.float32),
            pltpu.VMEM((B,), jnp.float32),
        ],
    )
    def pool(idx_hbm, table_hbm, out_hbm, idx_v, row_v, acc_v):
        wid = lax.axis_index("s") * nc + lax.axis_index("c")
        pltpu.sync_copy(idx_hbm, idx_v)
        for di in range(dims_per_w):
            d = wid * dims_per_w + di
            pltpu.sync_copy(table_hbm.at[d], row_v)

            def chunk(c, carry):
                base = c * 16
                acc = plsc.load_gather(row_v, [idx_v[pl.ds(base, 16)]])
                for k in range(1, CTX):
                    acc = acc + plsc.load_gather(
                        row_v, [idx_v[pl.ds(k * B + base, 16)]]
                    )
                acc_v[pl.ds(base, 16)] = acc
                return carry

            lax.fori_loop(0, B // 16, chunk, 0)
            pltpu.sync_copy(acc_v, out_hbm.at[d])

    def run(inputs, emb_table):
        idx = inputs.T.reshape(n_idx)
        emb_t = emb_table.T
        return pool(idx, emb_t)

    return run


def _make_pooling(B, CTX, D):
    info = plsc.get_sparse_core_info()
    nc, ns = info.num_cores, info.num_subcores
    nw = nc * ns
    rows_per_w = B // nw
    idx_per_w = rows_per_w * CTX
    n_chunks = -(-idx_per_w // 128)
    while idx_per_w % n_chunks or (idx_per_w // n_chunks) % 8:
        n_chunks += 1
    chunk = idx_per_w // n_chunks

    mesh = plsc.VectorSubcoreMesh(core_axis_name="c", subcore_axis_name="s")

    @functools.partial(
        pl.kernel,
        mesh=mesh,
        out_type=jax.ShapeDtypeStruct((B, D), jnp.float32),
        compiler_params=pltpu.CompilerParams(
            use_tc_tiling_on_sc=False, needs_layout_passes=False
        ),
        scratch_types=[
            pltpu.VMEM((n_chunks, chunk), jnp.int32),
            pltpu.VMEM((idx_per_w, D), jnp.float32),
            pltpu.VMEM((rows_per_w, D), jnp.float32),
            pltpu.SemaphoreType.DMA,
        ],
    )
    def pool(idx_hbm, table_hbm, out_hbm, idx_v, rows_v, acc_v, sem):
        wid = lax.axis_index("s") * nc + lax.axis_index("c")
        pltpu.sync_copy(idx_hbm.at[pl.ds(wid * n_chunks, n_chunks)], idx_v)
        copies = [
            pltpu.async_copy(
                table_hbm.at[idx_v.at[j]],
                rows_v.at[pl.ds(j * chunk, chunk)],
                sem,
            )
            for j in range(n_chunks)
        ]
        for cp in copies:
            cp.wait()

        def body(r, carry):
            base = r * CTX
            for c in range(D // 16):
                sl = pl.ds(c * 16, 16)
                acc = rows_v[base, sl]
                for k in range(1, CTX):
                    acc = acc + rows_v[base + k, sl]
                acc_v[r, sl] = acc
            return carry

        lax.fori_loop(0, rows_per_w, body, 0)
        pltpu.sync_copy(acc_v, out_hbm.at[pl.ds(wid * rows_per_w, rows_per_w)])

    def run(inputs, emb_table):
        idx = inputs.reshape(nw * n_chunks, chunk)
        return pool(idx, emb_table)

    return run


def _project(pooled_t, W, b, vb=2048):
    D, B = pooled_t.shape
    V = W.shape[0]
    w_t = W.T
    b2 = b.reshape(V, 1)

    def mm(w_ref, p_ref, b_ref, o_ref):
        o_ref[...] = (
            lax.dot_general(
                w_ref[...],
                p_ref[...],
                dimension_numbers=(((0,), (0,)), ((), ())),
                preferred_element_type=jnp.float32,
            )
            + b_ref[...]
        )

    out_t = pl.pallas_call(
        mm,
        grid=(pl.cdiv(V, vb),),
        in_specs=[
            pl.BlockSpec((D, vb), lambda v: (0, v)),
            pl.BlockSpec((D, B), lambda v: (0, 0)),
            pl.BlockSpec((vb, 1), lambda v: (v, 0)),
        ],
        out_specs=pl.BlockSpec((vb, B), lambda v: (v, 0)),
        out_shape=jax.ShapeDtypeStruct((V, B), jnp.float32),
    )(w_t, pooled_t, b2)
    return out_t.T


def kernel(inputs, emb_table, W, b):
    B, CTX = inputs.shape
    V, D = emb_table.shape
    pooled_t = _make_pooling_t(B, CTX, D, V)(inputs, emb_table)
    return _project(pooled_t, W, b)


# speedup vs baseline: 2.9647x; 1.2209x over previous
import functools

import jax
import jax.numpy as jnp
from jax import lax
from jax.experimental import pallas as pl
from jax.experimental.pallas import tpu as pltpu
from jax.experimental.pallas import tpu_sc as plsc


def _make_pooling_t(B, CTX, D, V):
    info = plsc.get_sparse_core_info()
    nc, ns = info.num_cores, info.num_subcores
    nw = nc * ns
    dims_per_w = D // nw
    n_idx = B * CTX

    mesh = plsc.VectorSubcoreMesh(core_axis_name="c", subcore_axis_name="s")

    @functools.partial(
        pl.kernel,
        mesh=mesh,
        out_type=jax.ShapeDtypeStruct((D, B), jnp.float32),
        compiler_params=pltpu.CompilerParams(
            use_tc_tiling_on_sc=False, needs_layout_passes=False
        ),
        scratch_types=[
            pltpu.VMEM((n_idx,), jnp.int32),
            pltpu.VMEM((V,), jnp.float32),
            pltpu.VMEM((B,), jnp.float32),
        ],
    )
    def pool(idx_hbm, table_hbm, out_hbm, idx_v, row_v, acc_v):
        wid = lax.axis_index("s") * nc + lax.axis_index("c")
        pltpu.sync_copy(idx_hbm, idx_v)
        for di in range(dims_per_w):
            d = wid * dims_per_w + di
            pltpu.sync_copy(table_hbm.at[d], row_v)

            def chunk(c, carry):
                base = c * 16
                acc = plsc.load_gather(row_v, [idx_v[pl.ds(base, 16)]])
                for k in range(1, CTX):
                    acc = acc + plsc.load_gather(
                        row_v, [idx_v[pl.ds(k * B + base, 16)]]
                    )
                acc_v[pl.ds(base, 16)] = acc
                return carry

            lax.fori_loop(0, B // 16, chunk, 0)
            pltpu.sync_copy(acc_v, out_hbm.at[d])

    def run(inputs, emb_table):
        idx = inputs.T.reshape(n_idx)
        emb_t = emb_table.T
        return pool(idx, emb_t)

    return run


def _make_pooling(B, CTX, D):
    info = plsc.get_sparse_core_info()
    nc, ns = info.num_cores, info.num_subcores
    nw = nc * ns
    rows_per_w = B // nw
    idx_per_w = rows_per_w * CTX
    n_chunks = -(-idx_per_w // 128)
    while idx_per_w % n_chunks or (idx_per_w // n_chunks) % 8:
        n_chunks += 1
    chunk = idx_per_w // n_chunks

    mesh = plsc.VectorSubcoreMesh(core_axis_name="c", subcore_axis_name="s")

    @functools.partial(
        pl.kernel,
        mesh=mesh,
        out_type=jax.ShapeDtypeStruct((B, D), jnp.float32),
        compiler_params=pltpu.CompilerParams(
            use_tc_tiling_on_sc=False, needs_layout_passes=False
        ),
        scratch_types=[
            pltpu.VMEM((n_chunks, chunk), jnp.int32),
            pltpu.VMEM((idx_per_w, D), jnp.float32),
            pltpu.VMEM((rows_per_w, D), jnp.float32),
            pltpu.SemaphoreType.DMA,
        ],
    )
    def pool(idx_hbm, table_hbm, out_hbm, idx_v, rows_v, acc_v, sem):
        wid = lax.axis_index("s") * nc + lax.axis_index("c")
        pltpu.sync_copy(idx_hbm.at[pl.ds(wid * n_chunks, n_chunks)], idx_v)
        copies = [
            pltpu.async_copy(
                table_hbm.at[idx_v.at[j]],
                rows_v.at[pl.ds(j * chunk, chunk)],
                sem,
            )
            for j in range(n_chunks)
        ]
        for cp in copies:
            cp.wait()

        def body(r, carry):
            base = r * CTX
            for c in range(D // 16):
                sl = pl.ds(c * 16, 16)
                acc = rows_v[base, sl]
                for k in range(1, CTX):
                    acc = acc + rows_v[base + k, sl]
                acc_v[r, sl] = acc
            return carry

        lax.fori_loop(0, rows_per_w, body, 0)
        pltpu.sync_copy(acc_v, out_hbm.at[pl.ds(wid * rows_per_w, rows_per_w)])

    def run(inputs, emb_table):
        idx = inputs.reshape(nw * n_chunks, chunk)
        return pool(idx, emb_table)

    return run


def _project(pooled_t, W, b, vb=2048):
    D, B = pooled_t.shape
    V = W.shape[0]
    w_t = W.T
    b2 = b.reshape(1, V)

    def mm(w_ref, p_ref, b_ref, o_ref):
        o_ref[...] = lax.dot_general(
            w_ref[...],
            p_ref[...],
            dimension_numbers=(((0,), (0,)), ((), ())),
            preferred_element_type=jnp.float32,
        ) + jnp.transpose(b_ref[...])

    out_t = pl.pallas_call(
        mm,
        grid=(pl.cdiv(V, vb),),
        in_specs=[
            pl.BlockSpec((D, vb), lambda v: (0, v)),
            pl.BlockSpec((D, B), lambda v: (0, 0)),
            pl.BlockSpec((1, vb), lambda v: (0, v)),
        ],
        out_specs=pl.BlockSpec((vb, B), lambda v: (v, 0)),
        out_shape=jax.ShapeDtypeStruct((V, B), jnp.float32),
    )(w_t, pooled_t, b2)
    return out_t.T


def kernel(inputs, emb_table, W, b):
    B, CTX = inputs.shape
    V, D = emb_table.shape
    pooled_t = _make_pooling_t(B, CTX, D, V)(inputs, emb_table)
    return _project(pooled_t, W, b)


# speedup vs baseline: 3.0022x; 1.0127x over previous
import functools

import jax
import jax.numpy as jnp
from jax import lax
from jax.experimental import pallas as pl
from jax.experimental.pallas import tpu as pltpu
from jax.experimental.pallas import tpu_sc as plsc


def _make_pooling_t(B, CTX, D, V):
    info = plsc.get_sparse_core_info()
    nc, ns = info.num_cores, info.num_subcores
    nw = nc * ns
    dims_per_w = D // nw
    n_idx = B * CTX

    mesh = plsc.VectorSubcoreMesh(core_axis_name="c", subcore_axis_name="s")

    @functools.partial(
        pl.kernel,
        mesh=mesh,
        out_type=jax.ShapeDtypeStruct((D, B), jnp.float32),
        compiler_params=pltpu.CompilerParams(
            use_tc_tiling_on_sc=False, needs_layout_passes=False
        ),
        scratch_types=[
            pltpu.VMEM((n_idx,), jnp.int32),
            pltpu.VMEM((V,), jnp.float32),
            pltpu.VMEM((B,), jnp.float32),
        ],
    )
    def pool(idx_hbm, table_hbm, out_hbm, idx_v, row_v, acc_v):
        wid = lax.axis_index("s") * nc + lax.axis_index("c")
        pltpu.sync_copy(idx_hbm, idx_v)
        for di in range(dims_per_w):
            d = wid * dims_per_w + di
            pltpu.sync_copy(table_hbm.at[d], row_v)

            def chunk(c, carry):
                base = c * 16
                acc = plsc.load_gather(row_v, [idx_v[pl.ds(base, 16)]])
                for k in range(1, CTX):
                    acc = acc + plsc.load_gather(
                        row_v, [idx_v[pl.ds(k * B + base, 16)]]
                    )
                acc_v[pl.ds(base, 16)] = acc
                return carry

            lax.fori_loop(0, B // 16, chunk, 0)
            pltpu.sync_copy(acc_v, out_hbm.at[d])

    def run(inputs, emb_table):
        idx = inputs.T.reshape(n_idx)
        emb_t = emb_table.T
        return pool(idx, emb_t)

    return run


def _make_pooling(B, CTX, D):
    info = plsc.get_sparse_core_info()
    nc, ns = info.num_cores, info.num_subcores
    nw = nc * ns
    rows_per_w = B // nw
    idx_per_w = rows_per_w * CTX
    n_chunks = -(-idx_per_w // 128)
    while idx_per_w % n_chunks or (idx_per_w // n_chunks) % 8:
        n_chunks += 1
    chunk = idx_per_w // n_chunks

    mesh = plsc.VectorSubcoreMesh(core_axis_name="c", subcore_axis_name="s")

    @functools.partial(
        pl.kernel,
        mesh=mesh,
        out_type=jax.ShapeDtypeStruct((B, D), jnp.float32),
        compiler_params=pltpu.CompilerParams(
            use_tc_tiling_on_sc=False, needs_layout_passes=False
        ),
        scratch_types=[
            pltpu.VMEM((n_chunks, chunk), jnp.int32),
            pltpu.VMEM((idx_per_w, D), jnp.float32),
            pltpu.VMEM((rows_per_w, D), jnp.float32),
            pltpu.SemaphoreType.DMA,
        ],
    )
    def pool(idx_hbm, table_hbm, out_hbm, idx_v, rows_v, acc_v, sem):
        wid = lax.axis_index("s") * nc + lax.axis_index("c")
        pltpu.sync_copy(idx_hbm.at[pl.ds(wid * n_chunks, n_chunks)], idx_v)
        copies = [
            pltpu.async_copy(
                table_hbm.at[idx_v.at[j]],
                rows_v.at[pl.ds(j * chunk, chunk)],
                sem,
            )
            for j in range(n_chunks)
        ]
        for cp in copies:
            cp.wait()

        def body(r, carry):
            base = r * CTX
            for c in range(D // 16):
                sl = pl.ds(c * 16, 16)
                acc = rows_v[base, sl]
                for k in range(1, CTX):
                    acc = acc + rows_v[base + k, sl]
                acc_v[r, sl] = acc
            return carry

        lax.fori_loop(0, rows_per_w, body, 0)
        pltpu.sync_copy(acc_v, out_hbm.at[pl.ds(wid * rows_per_w, rows_per_w)])

    def run(inputs, emb_table):
        idx = inputs.reshape(nw * n_chunks, chunk)
        return pool(idx, emb_table)

    return run


def _project(pooled_t, W, b, vb=4096):
    D, B = pooled_t.shape
    V = W.shape[0]
    w_t = W.T
    b2 = b.reshape(1, V)

    def mm(w_ref, p_ref, b_ref, o_ref):
        o_ref[...] = lax.dot_general(
            w_ref[...],
            p_ref[...],
            dimension_numbers=(((0,), (0,)), ((), ())),
            preferred_element_type=jnp.float32,
        ) + jnp.transpose(b_ref[...])

    out_t = pl.pallas_call(
        mm,
        grid=(pl.cdiv(V, vb),),
        in_specs=[
            pl.BlockSpec((D, vb), lambda v: (0, v)),
            pl.BlockSpec((D, B), lambda v: (0, 0)),
            pl.BlockSpec((1, vb), lambda v: (0, v)),
        ],
        out_specs=pl.BlockSpec((vb, B), lambda v: (v, 0)),
        out_shape=jax.ShapeDtypeStruct((V, B), jnp.float32),
    )(w_t, pooled_t, b2)
    return out_t.T


def kernel(inputs, emb_table, W, b):
    B, CTX = inputs.shape
    V, D = emb_table.shape
    pooled_t = _make_pooling_t(B, CTX, D, V)(inputs, emb_table)
    return _project(pooled_t, W, b)


# speedup vs baseline: 3.2435x; 1.0804x over previous
import functools

import jax
import jax.numpy as jnp
from jax import lax
from jax.experimental import pallas as pl
from jax.experimental.pallas import tpu as pltpu
from jax.experimental.pallas import tpu_sc as plsc


def _make_pooling_part(B, CTX, D, V):
    info = plsc.get_sparse_core_info()
    nc, ns = info.num_cores, info.num_subcores
    nw = nc * ns
    n_slabs = D // 8
    nq = nw // n_slabs
    n_idx = B * CTX
    lanes = 128
    n_chunks = 2 * nq
    full_w = (-(-V // (lanes * n_chunks))) * lanes
    last_lo = (n_chunks - 1) * full_w
    last_main = (V - last_lo) // lanes * lanes
    last_tail = V - last_lo - last_main

    mesh = plsc.VectorSubcoreMesh(core_axis_name="c", subcore_axis_name="s")

    @functools.partial(
        pl.kernel,
        mesh=mesh,
        out_type=jax.ShapeDtypeStruct((nq, D, B), jnp.float32),
        compiler_params=pltpu.CompilerParams(needs_layout_passes=False),
        scratch_types=[
            pltpu.VMEM((n_idx,), jnp.int32),
            pltpu.VMEM((8, full_w), jnp.float32),
            pltpu.VMEM((D, 128), jnp.float32),
            pltpu.VMEM((8, B), jnp.float32),
        ],
    )
    def pool(idx_hbm, table_hbm, tail_hbm, out_hbm, idx_v, slab_v, tail_v, acc_v):
        wid = lax.axis_index("s") * nc + lax.axis_index("c")
        q = wid % nq
        s = wid // nq
        pltpu.sync_copy(idx_hbm, idx_v)

        zero = jnp.zeros((16,), jnp.float32)

        def zbody(c, carry):
            base = c * 16
            for di in range(8):
                acc_v[di, pl.ds(base, 16)] = zero
            return carry

        lax.fori_loop(0, B // 16, zbody, 0)

        def gather_chunk(lo, width, ref, dbase):
            def body(c, carry):
                base = c * 16
                ivs = []
                for k in range(CTX):
                    iv = idx_v[pl.ds(k * B + base, 16)]
                    loc = iv - lo
                    m = (loc >= 0) & (loc < width)
                    loc = jnp.where(m, loc, 0)
                    ivs.append((loc, m))
                for di in range(8):
                    dvec = jnp.full((16,), di, jnp.int32) + dbase
                    acc = acc_v[di, pl.ds(base, 16)]
                    for loc, m in ivs:
                        g = plsc.load_gather(ref, [dvec, loc], mask=m)
                        acc = acc + jnp.where(m, g, 0.0)
                    acc_v[di, pl.ds(base, 16)] = acc
                return carry

            lax.fori_loop(0, B // 16, body, 0)

        for cid in range(n_chunks):
            lo = cid * full_w
            @pl.when(q == cid // 2)
            def _():
                if cid < n_chunks - 1:
                    pltpu.sync_copy(
                        table_hbm.at[pl.ds(s * 8, 8), pl.ds(lo, full_w)],
                        slab_v,
                    )
                    gather_chunk(lo, full_w, slab_v, 0)
                else:
                    pltpu.sync_copy(
                        table_hbm.at[pl.ds(s * 8, 8), pl.ds(lo, last_main)],
                        slab_v.at[pl.ds(0, 8), pl.ds(0, last_main)],
                    )
                    gather_chunk(lo, last_main, slab_v, 0)
                    if last_tail:
                        pltpu.sync_copy(tail_hbm, tail_v)
                        gather_chunk(lo + last_main, last_tail, tail_v, s * 8)

        pltpu.sync_copy(acc_v, out_hbm.at[q, pl.ds(s * 8, 8)])

    def run(inputs, emb_table):
        idx = inputs.T.reshape(n_idx)
        emb_t = emb_table.T
        tail_t = jnp.pad(
            emb_table[V - last_tail :].T, ((0, 0), (0, 128 - last_tail))
        )
        return pool(idx, emb_t, tail_t)

    return run, nq


def _project(part, W, b, vb=4096):
    nq, D, B = part.shape
    V = W.shape[0]
    w_t = W.T
    b2 = b.reshape(1, V)

    def mm(part_ref, w_ref, b_ref, o_ref):
        p = part_ref[0]
        for qq in range(1, nq):
            p = p + part_ref[qq]
        o_ref[...] = lax.dot_general(
            w_ref[...],
            p,
            dimension_numbers=(((0,), (0,)), ((), ())),
            preferred_element_type=jnp.float32,
        ) + jnp.transpose(b_ref[...])

    out_t = pl.pallas_call(
        mm,
        grid=(pl.cdiv(V, vb),),
        in_specs=[
            pl.BlockSpec((nq, D, B), lambda v: (0, 0, 0)),
            pl.BlockSpec((D, vb), lambda v: (0, v)),
            pl.BlockSpec((1, vb), lambda v: (0, v)),
        ],
        out_specs=pl.BlockSpec((vb, B), lambda v: (v, 0)),
        out_shape=jax.ShapeDtypeStruct((V, B), jnp.float32),
    )(part, w_t, b2)
    return out_t.T


def kernel(inputs, emb_table, W, b):
    B, CTX = inputs.shape
    V, D = emb_table.shape
    run, _ = _make_pooling_part(B, CTX, D, V)
    part = run(inputs, emb_table)
    return _project(part, W, b)


# speedup vs baseline: 3.3462x; 1.0316x over previous
import functools

import jax
import jax.numpy as jnp
from jax import lax
from jax.experimental import pallas as pl
from jax.experimental.pallas import tpu as pltpu
from jax.experimental.pallas import tpu_sc as plsc


def _make_pooling_part(B, CTX, D, V):
    info = plsc.get_sparse_core_info()
    nc, ns = info.num_cores, info.num_subcores
    nw = nc * ns
    n_slabs = D // 8
    nq = nw // n_slabs
    n_idx = B * CTX
    lanes = 128
    n_chunks = 2 * nq
    full_w = (-(-V // (lanes * n_chunks))) * lanes
    last_lo = (n_chunks - 1) * full_w
    last_main = (V - last_lo) // lanes * lanes
    last_tail = V - last_lo - last_main

    mesh = plsc.VectorSubcoreMesh(core_axis_name="c", subcore_axis_name="s")

    @functools.partial(
        pl.kernel,
        mesh=mesh,
        out_type=jax.ShapeDtypeStruct((nq, D, B), jnp.float32),
        compiler_params=pltpu.CompilerParams(needs_layout_passes=False),
        scratch_types=[
            pltpu.VMEM((n_idx,), jnp.int32),
            pltpu.VMEM((8, full_w), jnp.float32),
            pltpu.VMEM((8, B), jnp.float32),
        ],
    )
    def pool(idx_hbm, table_hbm, tail_hbm, out_hbm, idx_v, slab_v, acc_v):
        wid = lax.axis_index("s") * nc + lax.axis_index("c")
        q = wid % nq
        s = wid // nq
        pltpu.sync_copy(idx_hbm, idx_v)

        zero = jnp.zeros((16,), jnp.float32)

        def zbody(c, carry):
            base = c * 16
            for di in range(8):
                acc_v[di, pl.ds(base, 16)] = zero
            return carry

        lax.fori_loop(0, B // 16, zbody, 0)

        def gather_chunk(lo, width, ref, dbase):
            def body(c, carry):
                base = c * 16
                ivs = []
                for k in range(CTX):
                    iv = idx_v[pl.ds(k * B + base, 16)]
                    loc = iv - lo
                    m = (loc >= 0) & (loc < width)
                    loc = jnp.where(m, loc, 0)
                    ivs.append((loc, m))
                for di in range(8):
                    dvec = jnp.full((16,), di, jnp.int32) + dbase
                    acc = acc_v[di, pl.ds(base, 16)]
                    for loc, m in ivs:
                        g = plsc.load_gather(ref, [dvec, loc], mask=m)
                        acc = acc + jnp.where(m, g, 0.0)
                    acc_v[di, pl.ds(base, 16)] = acc
                return carry

            lax.fori_loop(0, B // 16, body, 0)

        for cid in range(n_chunks):
            lo = cid * full_w
            @pl.when(q == cid // 2)
            def _():
                if cid < n_chunks - 1:
                    pltpu.sync_copy(
                        table_hbm.at[pl.ds(s * 8, 8), pl.ds(lo, full_w)],
                        slab_v,
                    )
                    gather_chunk(lo, full_w, slab_v, 0)
                else:
                    pltpu.sync_copy(
                        table_hbm.at[pl.ds(s * 8, 8), pl.ds(lo, last_main)],
                        slab_v.at[pl.ds(0, 8), pl.ds(0, last_main)],
                    )
                    if last_tail:
                        pltpu.sync_copy(
                            tail_hbm.at[pl.ds(s * 8, 8)],
                            slab_v.at[pl.ds(0, 8), pl.ds(last_main, 128)],
                        )
                    gather_chunk(lo, last_main + last_tail, slab_v, 0)

        pltpu.sync_copy(acc_v, out_hbm.at[q, pl.ds(s * 8, 8)])

    def run(inputs, emb_table):
        idx = inputs.T.reshape(n_idx)
        emb_t = emb_table.T
        tail_t = jnp.pad(
            emb_table[V - last_tail :].T, ((0, 0), (0, 128 - last_tail))
        )
        return pool(idx, emb_t, tail_t)

    return run, nq


def _project(part, W, b, vb=4096):
    nq, D, B = part.shape
    V = W.shape[0]
    w_t = W.T
    b2 = b.reshape(1, V)

    def mm(part_ref, w_ref, b_ref, o_ref):
        p = part_ref[0]
        for qq in range(1, nq):
            p = p + part_ref[qq]
        o_ref[...] = lax.dot_general(
            w_ref[...],
            p,
            dimension_numbers=(((0,), (0,)), ((), ())),
            preferred_element_type=jnp.float32,
        ) + jnp.transpose(b_ref[...])

    out_t = pl.pallas_call(
        mm,
        grid=(pl.cdiv(V, vb),),
        in_specs=[
            pl.BlockSpec((nq, D, B), lambda v: (0, 0, 0)),
            pl.BlockSpec((D, vb), lambda v: (0, v)),
            pl.BlockSpec((1, vb), lambda v: (0, v)),
        ],
        out_specs=pl.BlockSpec((vb, B), lambda v: (v, 0)),
        out_shape=jax.ShapeDtypeStruct((V, B), jnp.float32),
    )(part, w_t, b2)
    return out_t.T


def kernel(inputs, emb_table, W, b):
    B, CTX = inputs.shape
    V, D = emb_table.shape
    run, _ = _make_pooling_part(B, CTX, D, V)
    part = run(inputs, emb_table)
    return _project(part, W, b)
